# Initial kernel scaffold; baseline (speedup 1.0000x reference)
import jax
import jax.numpy as jnp
from jax.experimental import pallas as pl


def kernel(x, table, W1, b1, W2, b2):
    raise NotImplementedError("write your pallas kernel here")


# speedup vs baseline: 6.3259x; 6.3259x over previous
import functools

import jax
import jax.numpy as jnp
import numpy as np
from jax import lax
from jax.experimental import pallas as pl
from jax.experimental.pallas import tpu as pltpu
from jax.experimental.pallas import tpu_sc as plsc

NC = 2
NS = 16
CHUNK = 128


def _make_pool_kernel(B, L, V, E):
    NW = NC * NS
    b_per_w = B // NW
    idx_per_w = b_per_w * L
    n_chunks = idx_per_w // CHUNK
    assert idx_per_w % CHUNK == 0 and n_chunks % 2 == 0
    b_per_sc = B // NC

    mesh = plsc.VectorSubcoreMesh(core_axis_name="c", subcore_axis_name="s",
                                  num_cores=NC, num_subcores=NS)

    @functools.partial(
        pl.kernel,
        out_type=jax.ShapeDtypeStruct((B, E), jnp.float32),
        mesh=mesh,
        compiler_params=pltpu.CompilerParams(use_tc_tiling_on_sc=False),
        scratch_types=[
            pltpu.VMEM((2, CHUNK), jnp.int32),
            pltpu.VMEM((2, CHUNK), jnp.int32),
            pltpu.VMEM((2, CHUNK, E), jnp.float32),
            pltpu.VMEM_SHARED((b_per_sc, E), jnp.float32),
            pltpu.SemaphoreType.DMA,
            pltpu.SemaphoreType.DMA,
        ],
    )
    def pool(x_hbm, dest_hbm, table_hbm, out_hbm,
             idx_v, dst_v, rows_v, pooled_s, sem0, sem1):
        c = lax.axis_index("c")
        s = lax.axis_index("s")
        w = c * NS + s
        base = w * idx_per_w

        zeros16 = jnp.zeros((16,), jnp.float32)

        def zrow(i, carry):
            for j in range(E // 16):
                rows_v[0, i, pl.ds(j * 16, 16)] = zeros16
            return carry

        lax.fori_loop(0, b_per_w, zrow, 0)
        pltpu.sync_copy(rows_v.at[0, pl.ds(0, b_per_w)],
                        pooled_s.at[pl.ds(s * b_per_w, b_per_w)])
        plsc.subcore_barrier()

        def load_idx(g, slot):
            pltpu.sync_copy(x_hbm.at[pl.ds(base + g * CHUNK, CHUNK)],
                            idx_v.at[slot])
            pltpu.sync_copy(dest_hbm.at[pl.ds(base + g * CHUNK, CHUNK)],
                            dst_v.at[slot])

        def start_gather(slot, sem):
            pltpu.async_copy(table_hbm.at[idx_v.at[slot]], rows_v.at[slot], sem)

        def wait_gather(slot, sem):
            pltpu.make_async_copy(table_hbm.at[idx_v.at[slot]],
                                  rows_v.at[slot], sem).wait()

        def scatter_add(slot):
            pltpu.sync_copy(rows_v.at[slot], pooled_s.at[dst_v.at[slot]],
                            add=True)

        load_idx(0, 0)
        start_gather(0, sem0)

        def step(i, carry):
            g = i * 2
            load_idx(g + 1, 1)
            start_gather(1, sem1)
            wait_gather(0, sem0)
            scatter_add(0)

            @pl.when(g + 2 < n_chunks)
            def _():
                load_idx(g + 2, 0)
                start_gather(0, sem0)

            wait_gather(1, sem1)
            scatter_add(1)
            return carry

        lax.fori_loop(0, n_chunks // 2, step, 0)

        plsc.subcore_barrier()
        pltpu.sync_copy(pooled_s.at[pl.ds(s * b_per_w, b_per_w)],
                        out_hbm.at[pl.ds(w * b_per_w, b_per_w)])

    return pool


def _ffn(pooled_sum, W1, b1, W2, b2, L):
    B, E = pooled_sum.shape
    HID = W1.shape[0]
    NCLS = W2.shape[0]
    blk = 512
    inv_l = np.float32(1.0 / L)

    def body(p_ref, w1_ref, b1_ref, w2_ref, b2_ref, o_ref):
        p = p_ref[...] * inv_l
        h = lax.dot_general(p, w1_ref[...], (((1,), (1,)), ((), ())),
                            preferred_element_type=jnp.float32,
                            precision=lax.Precision.HIGHEST)
        h = jnp.maximum(h + b1_ref[...], 0.0)
        o = lax.dot_general(h, w2_ref[...], (((1,), (1,)), ((), ())),
                            preferred_element_type=jnp.float32,
                            precision=lax.Precision.HIGHEST)
        o_ref[...] = o + b2_ref[...]

    return pl.pallas_call(
        body,
        grid=(B // blk,),
        in_specs=[
            pl.BlockSpec((blk, E), lambda i: (i, 0)),
            pl.BlockSpec((HID, E), lambda i: (0, 0)),
            pl.BlockSpec((1, HID), lambda i: (0, 0)),
            pl.BlockSpec((NCLS, HID), lambda i: (0, 0)),
            pl.BlockSpec((1, NCLS), lambda i: (0, 0)),
        ],
        out_specs=pl.BlockSpec((blk, NCLS), lambda i: (i, 0)),
        out_shape=jax.ShapeDtypeStruct((B, NCLS), jnp.float32),
    )(pooled_sum, W1, b1.reshape(1, HID), W2, b2.reshape(1, NCLS))


def kernel(x, table, W1, b1, W2, b2):
    B, L = x.shape
    V, E = table.shape
    x_flat = x.reshape(-1).astype(jnp.int32)
    b_per_sc = B // NC
    dest = jnp.asarray(
        np.repeat(np.arange(B, dtype=np.int32) % (B // NC), L))
    pool = _make_pool_kernel(B, L, V, E)
    pooled_sum = pool(x_flat, dest, table)
    return _ffn(pooled_sum, W1, b1, W2, b2, L)


# speedup vs baseline: 8.2260x; 1.3004x over previous
import functools

import jax
import jax.numpy as jnp
import numpy as np
from jax import lax
from jax.experimental import pallas as pl
from jax.experimental.pallas import tpu as pltpu
from jax.experimental.pallas import tpu_sc as plsc

NC = 2
NS = 16
NBUF = 4


def _make_pool_kernel(B, L, V, E):
    NW = NC * NS
    b_per_w = B // NW
    b_per_sc = B // NC
    assert b_per_w % NBUF == 0

    mesh = plsc.VectorSubcoreMesh(core_axis_name="c", subcore_axis_name="s",
                                  num_cores=NC, num_subcores=NS)

    @functools.partial(
        pl.kernel,
        out_type=jax.ShapeDtypeStruct((B, E), jnp.float32),
        mesh=mesh,
        compiler_params=pltpu.CompilerParams(use_tc_tiling_on_sc=False),
        scratch_types=[
            pltpu.VMEM((b_per_w, L), jnp.int32),
            pltpu.VMEM((b_per_w, L), jnp.int32),
            pltpu.VMEM((NBUF, L, E), jnp.float32),
            pltpu.VMEM((b_per_w, E), jnp.float32),
            pltpu.VMEM_SHARED((b_per_sc, E), jnp.float32),
            [pltpu.SemaphoreType.DMA] * NBUF,
        ],
    )
    def pool(x_hbm, dest_hbm, table_hbm, out_hbm,
             idx_all, dst_all, rows_v, zbuf, pooled_s, sems):
        c = lax.axis_index("c")
        s = lax.axis_index("s")
        w = c * NS + s
        row0 = w * b_per_w

        pltpu.sync_copy(x_hbm.at[pl.ds(row0, b_per_w)], idx_all)
        pltpu.sync_copy(dest_hbm.at[pl.ds(row0, b_per_w)], dst_all)

        zeros16 = jnp.zeros((16,), jnp.float32)

        def zrow(i, carry):
            for j in range(E // 16):
                zbuf[i, pl.ds(j * 16, 16)] = zeros16
            return carry

        lax.fori_loop(0, b_per_w, zrow, 0)
        pltpu.sync_copy(zbuf, pooled_s.at[pl.ds(s * b_per_w, b_per_w)])

        def start_gather(b, slot):
            pltpu.async_copy(table_hbm.at[idx_all.at[b]], rows_v.at[slot],
                             sems[slot])

        def wait_gather(b, slot):
            pltpu.make_async_copy(table_hbm.at[idx_all.at[b]],
                                  rows_v.at[slot], sems[slot]).wait()

        def scatter_add(b, slot):
            pltpu.sync_copy(rows_v.at[slot], pooled_s.at[dst_all.at[b]],
                            add=True)

        for k in range(NBUF):
            start_gather(k, k)

        def step(i, carry):
            g = i * NBUF
            for k in range(NBUF):
                b = g + k
                wait_gather(b, k)
                scatter_add(b, k)

                @pl.when(b + NBUF < b_per_w)
                def _():
                    start_gather(b + NBUF, k)
            return carry

        lax.fori_loop(0, b_per_w // NBUF, step, 0)

        pltpu.sync_copy(pooled_s.at[pl.ds(s * b_per_w, b_per_w)],
                        out_hbm.at[pl.ds(row0, b_per_w)])

    return pool


def _ffn(pooled_sum, W1, b1, W2, b2, L):
    B, E = pooled_sum.shape
    HID = W1.shape[0]
    NCLS = W2.shape[0]
    blk = 512
    inv_l = np.float32(1.0 / L)

    def body(p_ref, w1_ref, b1_ref, w2_ref, b2_ref, o_ref):
        p = p_ref[...] * inv_l
        h = lax.dot_general(p, w1_ref[...], (((1,), (1,)), ((), ())),
                            preferred_element_type=jnp.float32)
        h = jnp.maximum(h + b1_ref[...], 0.0)
        o = lax.dot_general(h, w2_ref[...], (((1,), (1,)), ((), ())),
                            preferred_element_type=jnp.float32)
        o_ref[...] = o + b2_ref[...]

    return pl.pallas_call(
        body,
        grid=(B // blk,),
        in_specs=[
            pl.BlockSpec((blk, E), lambda i: (i, 0)),
            pl.BlockSpec((HID, E), lambda i: (0, 0)),
            pl.BlockSpec((1, HID), lambda i: (0, 0)),
            pl.BlockSpec((NCLS, HID), lambda i: (0, 0)),
            pl.BlockSpec((1, NCLS), lambda i: (0, 0)),
        ],
        out_specs=pl.BlockSpec((blk, NCLS), lambda i: (i, 0)),
        out_shape=jax.ShapeDtypeStruct((B, NCLS), jnp.float32),
    )(pooled_sum, W1, b1.reshape(1, HID), W2, b2.reshape(1, NCLS))


def kernel(x, table, W1, b1, W2, b2):
    B, L = x.shape
    V, E = table.shape
    dest = jnp.asarray(
        np.broadcast_to((np.arange(B, dtype=np.int32) % (B // NC))[:, None],
                        (B, L)))
    pool = _make_pool_kernel(B, L, V, E)
    pooled_sum = pool(x.astype(jnp.int32), dest, table)
    return _ffn(pooled_sum, W1, b1, W2, b2, L)
